# baseline (speedup 1.0000x reference)
import functools

import jax
import jax.numpy as jnp
from jax import lax
from jax.experimental import pallas as pl
from jax.experimental.pallas import tpu as pltpu
from jax.experimental.pallas import tpu_sc as plsc

_H = 768
_F = 2048
_E = 8
_T = 2048
_A = 2 * _T
_BT = 256
_NBMAX = _A // _BT + _E
_NP = _NBMAX * _BT
_NMETA = 32

_NW = 32
_APW = _A // _NW
_TPW = _T // _NW

_INV_SQRT2 = 0.7071067811865476


def _router_body(x_ref, wr_ref, br_ref, pos_ref, gates_ref, bmeta_ref,
                 csum_ref):
    x = x_ref[...]
    logits = jnp.dot(x, wr_ref[...], preferred_element_type=jnp.float32)
    logits = logits + br_ref[0, :]
    logits = logits - jnp.max(logits, axis=-1, keepdims=True)
    p = jnp.exp(logits)
    p = p / jnp.sum(p, axis=-1, keepdims=True)

    ids = lax.broadcasted_iota(jnp.int32, (_T, _E), 1)
    i1 = jnp.argmax(p, axis=-1)
    m1 = jnp.max(p, axis=-1, keepdims=True)
    oh1 = ids == i1[:, None]
    p2 = jnp.where(oh1, -jnp.inf, p)
    i2 = jnp.argmax(p2, axis=-1)
    m2 = jnp.max(p2, axis=-1, keepdims=True)
    oh2 = ids == i2[:, None]
    denom = m1 + m2
    gates_ref[0, :] = (m1 / denom)[:, 0]
    gates_ref[1, :] = (m2 / denom)[:, 0]

    maskcat = jnp.concatenate(
        [oh1.astype(jnp.bfloat16), oh2.astype(jnp.bfloat16)], axis=1)
    ti = lax.broadcasted_iota(jnp.int32, (128, 128), 0)
    tj = lax.broadcasted_iota(jnp.int32, (128, 128), 1)
    ltri = (ti > tj).astype(jnp.bfloat16)
    run8 = jnp.zeros((1, 2 * _E), jnp.float32)
    for i in range(_T // 128):
        chunk = maskcat[i * 128:(i + 1) * 128, :]
        c = jnp.dot(ltri, chunk, preferred_element_type=jnp.float32)
        csum_ref[pl.ds(i * 128, 128), :] = c + run8
        run8 = run8 + jnp.sum(chunk.astype(jnp.float32), axis=0,
                              keepdims=True)
    csum = csum_ref[...]
    total0 = run8[:, :_E]
    total1 = run8[:, _E:]
    rank0 = csum[:, :_E]
    rank1 = csum[:, _E:] + total0

    count = total0 + total1
    padded = jnp.ceil(count / _BT) * _BT
    ei = lax.broadcasted_iota(jnp.int32, (_E, _E), 0)
    ej = lax.broadcasted_iota(jnp.int32, (_E, _E), 1)
    utri = (ei < ej).astype(jnp.float32)
    pstart = jnp.dot(padded, utri,
                     preferred_element_type=jnp.float32)

    dest0 = jnp.sum(jnp.where(oh1, pstart + rank0, 0.0), axis=1)
    dest1 = jnp.sum(jnp.where(oh2, pstart + rank1, 0.0), axis=1)
    pos_ref[0, :] = dest0.astype(jnp.int32)
    pos_ref[1, :] = dest1.astype(jnp.int32)

    pend = (pstart + padded)[0, :]
    bidx = lax.broadcasted_iota(jnp.int32, (1, _NMETA), 1).astype(jnp.float32)
    cmp = (pend[:, None] <= bidx * _BT).astype(jnp.float32)
    be_raw = jnp.sum(cmp, axis=0, keepdims=True)
    nb = jnp.sum(padded) * (1.0 / _BT)
    lastexp = jnp.sum(jnp.where(bidx == nb - 1.0, be_raw, 0.0))
    bexp = jnp.minimum(be_raw, lastexp)
    meta = jnp.where(bidx < _NBMAX, bexp, nb)
    bmeta_ref[...] = meta.astype(jnp.int32)


def _run_router(x, Wr, br):
    return pl.pallas_call(
        _router_body,
        in_specs=[
            pl.BlockSpec((_T, _H), lambda: (0, 0)),
            pl.BlockSpec((_H, _E), lambda: (0, 0)),
            pl.BlockSpec((1, _E), lambda: (0, 0)),
        ],
        out_specs=[
            pl.BlockSpec((2, _T), lambda: (0, 0)),
            pl.BlockSpec((2, _T), lambda: (0, 0)),
            pl.BlockSpec((1, _NMETA), lambda: (0, 0)),
        ],
        out_shape=[
            jax.ShapeDtypeStruct((2, _T), jnp.int32),
            jax.ShapeDtypeStruct((2, _T), jnp.float32),
            jax.ShapeDtypeStruct((1, _NMETA), jnp.int32),
        ],
        scratch_shapes=[pltpu.VMEM((_T, 2 * _E), jnp.float32)],
    )(x, Wr, br.reshape(1, _E))


_BCH = 4
_BCR = _APW // _BCH


def _dispatch_body(x_hbm, pos_hbm, xs_hbm, idx_v, rows_v,
                   sem_i, sem_in, sem_out):
    c = lax.axis_index("c")
    s = lax.axis_index("s")
    wid = s * 2 + c
    k = wid // (_NW // 2)
    tokbase = (wid % (_NW // 2)) * _APW
    ics = []
    for ch in range(_BCH):
        ics.append(pltpu.async_copy(
            pos_hbm.at[k, pl.ds(tokbase + ch * _BCR, _BCR)],
            idx_v.at[ch], sem_i))
    loads = []
    for ch in range(_BCH):
        loads.append(pltpu.async_copy(
            x_hbm.at[pl.ds(tokbase + ch * _BCR, _BCR)],
            rows_v.at[pl.ds(ch * _BCR, _BCR)], sem_in[ch]))
    for ic in ics:
        ic.wait()
    scatters = []
    for ch in range(_BCH):
        loads[ch].wait()
        scatters.append(pltpu.async_copy(
            rows_v.at[pl.ds(ch * _BCR, _BCR)],
            xs_hbm.at[idx_v.at[ch]], sem_out))
    for sc in scatters:
        sc.wait()


def _run_dispatch(x, pos2d):
    mesh = plsc.VectorSubcoreMesh(core_axis_name="c", subcore_axis_name="s")
    f = pl.kernel(
        _dispatch_body,
        mesh=mesh,
        out_type=jax.ShapeDtypeStruct((_NP, _H), jnp.float32),
        scratch_types=[
            pltpu.VMEM((_BCH, _BCR), jnp.int32),
            pltpu.VMEM((_APW, _H), jnp.float32),
            pltpu.SemaphoreType.DMA,
            [pltpu.SemaphoreType.DMA] * _BCH,
            pltpu.SemaphoreType.DMA,
        ],
    )
    return f(x, pos2d)


def _ffn_body(bmeta_ref, x_ref, w1_ref, b1_ref, w2_ref, b2_ref, y_ref,
              w1b_ref, w2b_ref):
    b = pl.program_id(0)
    nb = bmeta_ref[0, _NBMAX]
    prev = bmeta_ref[0, jnp.maximum(b - 1, 0)]
    changed = jnp.logical_or(b == 0, prev != bmeta_ref[0, b])

    @pl.when(jnp.logical_and(changed, b < nb))
    def _cast():
        w1b_ref[...] = w1_ref[0].astype(jnp.bfloat16)
        w2b_ref[...] = w2_ref[0].astype(jnp.bfloat16)

    @pl.when(b < nb)
    def _compute():
        xb = x_ref[...].astype(jnp.bfloat16)
        h = jnp.dot(xb, w1b_ref[...], preferred_element_type=jnp.float32)
        h = h + b1_ref[0]
        h = 0.5 * h * (1.0 + lax.erf(h * _INV_SQRT2))
        y = jnp.dot(h.astype(jnp.bfloat16), w2b_ref[...],
                    preferred_element_type=jnp.float32)
        y_ref[...] = y + b2_ref[0]


def _run_ffn(bmeta, xs, w1b, b1r, w2b, b2r):
    grid_spec = pltpu.PrefetchScalarGridSpec(
        num_scalar_prefetch=1,
        grid=(_NBMAX,),
        in_specs=[
            pl.BlockSpec((_BT, _H), lambda b, m: (b, 0)),
            pl.BlockSpec((1, _H, _F), lambda b, m: (m[0, b], 0, 0)),
            pl.BlockSpec((1, 1, _F), lambda b, m: (m[0, b], 0, 0)),
            pl.BlockSpec((1, _F, _H), lambda b, m: (m[0, b], 0, 0)),
            pl.BlockSpec((1, 1, _H), lambda b, m: (m[0, b], 0, 0)),
        ],
        out_specs=pl.BlockSpec((_BT, _H), lambda b, m: (b, 0)),
        scratch_shapes=[
            pltpu.VMEM((_H, _F), jnp.bfloat16),
            pltpu.VMEM((_F, _H), jnp.bfloat16),
        ],
    )
    return pl.pallas_call(
        _ffn_body,
        grid_spec=grid_spec,
        out_shape=jax.ShapeDtypeStruct((_NP, _H), jnp.float32),
        compiler_params=pltpu.CompilerParams(
            dimension_semantics=("arbitrary",),
        ),
    )(bmeta, xs, w1b, b1r, w2b, b2r)


def _combine_body(y_hbm, pos_hbm, gates_hbm, out_hbm,
                  idx0, idx1, g_vm, r0, r1, sem, sem0, sem1, semw):
    c = lax.axis_index("c")
    s = lax.axis_index("s")
    wid = s * 2 + c
    base = wid * _TPW
    c0 = pltpu.async_copy(pos_hbm.at[0, pl.ds(base, _TPW)], idx0, sem)
    c1 = pltpu.async_copy(pos_hbm.at[1, pl.ds(base, _TPW)], idx1, sem)
    c2 = pltpu.async_copy(gates_hbm.at[0, pl.ds(base, _TPW)], g_vm.at[0],
                          sem)
    c3 = pltpu.async_copy(gates_hbm.at[1, pl.ds(base, _TPW)], g_vm.at[1],
                          sem)
    c0.wait(); c1.wait(); c2.wait(); c3.wait()

    half = _TPW // 2
    gsems = (sem0, sem1)
    gathers = []
    for ch in range(2):
        hsl = pl.ds(ch * half, half)
        gathers.append((
            pltpu.async_copy(y_hbm.at[idx0.at[hsl]], r0.at[hsl], gsems[ch]),
            pltpu.async_copy(y_hbm.at[idx1.at[hsl]], r1.at[hsl], gsems[ch]),
        ))

    def body(blk, _):
        gv0 = g_vm[0, pl.ds(blk * 16, 16)]
        gv1 = g_vm[1, pl.ds(blk * 16, 16)]
        for ii in range(16):
            g0 = gv0[ii]
            g1 = gv1[ii]
            row = blk * 16 + ii
            for j in range(_H // 16):
                sl = pl.ds(j * 16, 16)
                r0[row, sl] = r0[row, sl] * g0 + r1[row, sl] * g1
        return 0

    writes = []
    nblk = half // 16
    for ch in range(2):
        ga, gb = gathers[ch]
        ga.wait()
        gb.wait()
        lax.fori_loop(ch * nblk, (ch + 1) * nblk, body, 0)
        hsl = pl.ds(ch * half, half)
        writes.append(pltpu.async_copy(
            r0.at[hsl], out_hbm.at[pl.ds(base + ch * half, half)], semw))
    for wcp in writes:
        wcp.wait()


def _run_combine(ys, pos, gates):
    mesh = plsc.VectorSubcoreMesh(core_axis_name="c", subcore_axis_name="s")
    f = pl.kernel(
        _combine_body,
        mesh=mesh,
        out_type=jax.ShapeDtypeStruct((_T, _H), jnp.float32),
        scratch_types=[
            pltpu.VMEM((_TPW,), jnp.int32),
            pltpu.VMEM((_TPW,), jnp.int32),
            pltpu.VMEM((2, _TPW), jnp.float32),
            pltpu.VMEM((_TPW, _H), jnp.float32),
            pltpu.VMEM((_TPW, _H), jnp.float32),
            pltpu.SemaphoreType.DMA,
            pltpu.SemaphoreType.DMA,
            pltpu.SemaphoreType.DMA,
            pltpu.SemaphoreType.DMA,
        ],
    )
    return f(ys, pos, gates)


def kernel(hidden_states, Wr, br, W1, b1, W2, b2):
    B, S, Hd = hidden_states.shape
    x = hidden_states.reshape(S, Hd)
    w1b = W1
    w2b = W2
    b1r = b1.reshape(_E, 1, _F)
    b2r = b2.reshape(_E, 1, _H)

    pos, gates, bmeta = _run_router(x, Wr, br)
    xs = _run_dispatch(x, pos)
    ys = _run_ffn(bmeta, xs, w1b, b1r, w2b, b2r)
    out = _run_combine(ys, pos, gates)
    return out.reshape(B, S, Hd)


# speedup vs baseline: 1.0344x; 1.0344x over previous
import functools

import jax
import jax.numpy as jnp
from jax import lax
from jax.experimental import pallas as pl
from jax.experimental.pallas import tpu as pltpu
from jax.experimental.pallas import tpu_sc as plsc

_H = 768
_F = 2048
_E = 8
_T = 2048
_A = 2 * _T
_BT = 256
_NBMAX = _A // _BT + _E
_NP = _NBMAX * _BT
_NMETA = 32

_NW = 32
_APW = _A // _NW
_TPW = _T // _NW

_INV_SQRT2 = 0.7071067811865476


def _router_body(x_ref, wr_ref, br_ref, pos_ref, gates_ref, bmeta_ref,
                 csum_ref):
    x = x_ref[...]
    logits = jnp.dot(x, wr_ref[...], preferred_element_type=jnp.float32)
    logits = logits + br_ref[0, :]
    logits = logits - jnp.max(logits, axis=-1, keepdims=True)
    p = jnp.exp(logits)
    p = p / jnp.sum(p, axis=-1, keepdims=True)

    ids = lax.broadcasted_iota(jnp.int32, (_T, _E), 1)
    i1 = jnp.argmax(p, axis=-1)
    m1 = jnp.max(p, axis=-1, keepdims=True)
    oh1 = ids == i1[:, None]
    p2 = jnp.where(oh1, -jnp.inf, p)
    i2 = jnp.argmax(p2, axis=-1)
    m2 = jnp.max(p2, axis=-1, keepdims=True)
    oh2 = ids == i2[:, None]
    denom = m1 + m2
    gates_ref[0, :] = (m1 / denom)[:, 0]
    gates_ref[1, :] = (m2 / denom)[:, 0]

    maskcat = jnp.concatenate(
        [oh1.astype(jnp.bfloat16), oh2.astype(jnp.bfloat16)], axis=1)
    ti = lax.broadcasted_iota(jnp.int32, (128, 128), 0)
    tj = lax.broadcasted_iota(jnp.int32, (128, 128), 1)
    ltri = (ti > tj).astype(jnp.bfloat16)
    run8 = jnp.zeros((1, 2 * _E), jnp.float32)
    for i in range(_T // 128):
        chunk = maskcat[i * 128:(i + 1) * 128, :]
        c = jnp.dot(ltri, chunk, preferred_element_type=jnp.float32)
        csum_ref[pl.ds(i * 128, 128), :] = c + run8
        run8 = run8 + jnp.sum(chunk.astype(jnp.float32), axis=0,
                              keepdims=True)
    csum = csum_ref[...]
    total0 = run8[:, :_E]
    total1 = run8[:, _E:]
    rank0 = csum[:, :_E]
    rank1 = csum[:, _E:] + total0

    count = total0 + total1
    padded = jnp.ceil(count / _BT) * _BT
    ei = lax.broadcasted_iota(jnp.int32, (_E, _E), 0)
    ej = lax.broadcasted_iota(jnp.int32, (_E, _E), 1)
    utri = (ei < ej).astype(jnp.float32)
    pstart = jnp.dot(padded, utri,
                     preferred_element_type=jnp.float32)

    dest0 = jnp.sum(jnp.where(oh1, pstart + rank0, 0.0), axis=1)
    dest1 = jnp.sum(jnp.where(oh2, pstart + rank1, 0.0), axis=1)
    pos_ref[0, :] = dest0.astype(jnp.int32)
    pos_ref[1, :] = dest1.astype(jnp.int32)

    pend = (pstart + padded)[0, :]
    bidx = lax.broadcasted_iota(jnp.int32, (1, _NMETA), 1).astype(jnp.float32)
    cmp = (pend[:, None] <= bidx * _BT).astype(jnp.float32)
    be_raw = jnp.sum(cmp, axis=0, keepdims=True)
    nb = jnp.sum(padded) * (1.0 / _BT)
    lastexp = jnp.sum(jnp.where(bidx == nb - 1.0, be_raw, 0.0))
    bexp = jnp.minimum(be_raw, lastexp)
    meta = jnp.where(bidx < _NBMAX, bexp, nb)
    bmeta_ref[...] = meta.astype(jnp.int32)


def _run_router(x, Wr, br):
    return pl.pallas_call(
        _router_body,
        in_specs=[
            pl.BlockSpec((_T, _H), lambda: (0, 0)),
            pl.BlockSpec((_H, _E), lambda: (0, 0)),
            pl.BlockSpec((1, _E), lambda: (0, 0)),
        ],
        out_specs=[
            pl.BlockSpec((2, _T), lambda: (0, 0)),
            pl.BlockSpec((2, _T), lambda: (0, 0)),
            pl.BlockSpec((1, _NMETA), lambda: (0, 0)),
        ],
        out_shape=[
            jax.ShapeDtypeStruct((2, _T), jnp.int32),
            jax.ShapeDtypeStruct((2, _T), jnp.float32),
            jax.ShapeDtypeStruct((1, _NMETA), jnp.int32),
        ],
        scratch_shapes=[pltpu.VMEM((_T, 2 * _E), jnp.float32)],
    )(x, Wr, br.reshape(1, _E))


_BCH = 4
_BCR = _APW // _BCH


def _dispatch_body(x_hbm, pos_hbm, xs_hbm, idx_v, rows_v,
                   sem_i, sem_in, sem_out):
    c = lax.axis_index("c")
    s = lax.axis_index("s")
    wid = s * 2 + c
    k = wid // (_NW // 2)
    tokbase = (wid % (_NW // 2)) * _APW
    ics = []
    for ch in range(_BCH):
        ics.append(pltpu.async_copy(
            pos_hbm.at[k, pl.ds(tokbase + ch * _BCR, _BCR)],
            idx_v.at[ch], sem_i))
    loads = []
    for ch in range(_BCH):
        loads.append(pltpu.async_copy(
            x_hbm.at[pl.ds(tokbase + ch * _BCR, _BCR)],
            rows_v.at[pl.ds(ch * _BCR, _BCR)], sem_in[ch]))
    for ic in ics:
        ic.wait()
    scatters = []
    for ch in range(_BCH):
        loads[ch].wait()
        scatters.append(pltpu.async_copy(
            rows_v.at[pl.ds(ch * _BCR, _BCR)],
            xs_hbm.at[idx_v.at[ch]], sem_out))
    for sc in scatters:
        sc.wait()


def _run_dispatch(x, pos2d):
    mesh = plsc.VectorSubcoreMesh(core_axis_name="c", subcore_axis_name="s")
    f = pl.kernel(
        _dispatch_body,
        mesh=mesh,
        out_type=jax.ShapeDtypeStruct((_NP, _H), jnp.float32),
        scratch_types=[
            pltpu.VMEM((_BCH, _BCR), jnp.int32),
            pltpu.VMEM((_APW, _H), jnp.float32),
            pltpu.SemaphoreType.DMA,
            [pltpu.SemaphoreType.DMA] * _BCH,
            pltpu.SemaphoreType.DMA,
        ],
    )
    return f(x, pos2d)


def _ffn_body(bmeta_ref, x_ref, w1_ref, b1_ref, w2_ref, b2_ref, y_ref):
    b = pl.program_id(0)
    nb = bmeta_ref[0, _NBMAX]

    @pl.when(b < nb)
    def _compute():
        xb = x_ref[...].astype(jnp.bfloat16)
        h = jnp.dot(xb, w1_ref[0].astype(jnp.bfloat16),
                    preferred_element_type=jnp.float32)
        h = h + b1_ref[0]
        h = 0.5 * h * (1.0 + lax.erf(h * _INV_SQRT2))
        y = jnp.dot(h.astype(jnp.bfloat16), w2_ref[0].astype(jnp.bfloat16),
                    preferred_element_type=jnp.float32)
        y_ref[...] = y + b2_ref[0]


def _run_ffn(bmeta, xs, w1b, b1r, w2b, b2r):
    grid_spec = pltpu.PrefetchScalarGridSpec(
        num_scalar_prefetch=1,
        grid=(_NBMAX,),
        in_specs=[
            pl.BlockSpec((_BT, _H), lambda b, m: (b, 0)),
            pl.BlockSpec((1, _H, _F), lambda b, m: (m[0, b], 0, 0)),
            pl.BlockSpec((1, 1, _F), lambda b, m: (m[0, b], 0, 0)),
            pl.BlockSpec((1, _F, _H), lambda b, m: (m[0, b], 0, 0)),
            pl.BlockSpec((1, 1, _H), lambda b, m: (m[0, b], 0, 0)),
        ],
        out_specs=pl.BlockSpec((_BT, _H), lambda b, m: (b, 0)),
    )
    return pl.pallas_call(
        _ffn_body,
        grid_spec=grid_spec,
        out_shape=jax.ShapeDtypeStruct((_NP, _H), jnp.float32),
        compiler_params=pltpu.CompilerParams(
            dimension_semantics=("arbitrary",),
        ),
    )(bmeta, xs, w1b, b1r, w2b, b2r)


def _combine_body(y_hbm, pos_hbm, gates_hbm, out_hbm,
                  idx0, idx1, g_vm, r0, r1, sem, sem0, sem1, semw):
    c = lax.axis_index("c")
    s = lax.axis_index("s")
    wid = s * 2 + c
    base = wid * _TPW
    c0 = pltpu.async_copy(pos_hbm.at[0, pl.ds(base, _TPW)], idx0, sem)
    c1 = pltpu.async_copy(pos_hbm.at[1, pl.ds(base, _TPW)], idx1, sem)
    c2 = pltpu.async_copy(gates_hbm.at[0, pl.ds(base, _TPW)], g_vm.at[0],
                          sem)
    c3 = pltpu.async_copy(gates_hbm.at[1, pl.ds(base, _TPW)], g_vm.at[1],
                          sem)
    c0.wait(); c1.wait(); c2.wait(); c3.wait()

    half = _TPW // 2
    gsems = (sem0, sem1)
    gathers = []
    for ch in range(2):
        hsl = pl.ds(ch * half, half)
        gathers.append((
            pltpu.async_copy(y_hbm.at[idx0.at[hsl]], r0.at[hsl], gsems[ch]),
            pltpu.async_copy(y_hbm.at[idx1.at[hsl]], r1.at[hsl], gsems[ch]),
        ))

    def body(blk, _):
        gv0 = g_vm[0, pl.ds(blk * 16, 16)]
        gv1 = g_vm[1, pl.ds(blk * 16, 16)]
        for ii in range(16):
            g0 = gv0[ii]
            g1 = gv1[ii]
            row = blk * 16 + ii
            for j in range(_H // 16):
                sl = pl.ds(j * 16, 16)
                r0[row, sl] = r0[row, sl] * g0 + r1[row, sl] * g1
        return 0

    writes = []
    nblk = half // 16
    for ch in range(2):
        ga, gb = gathers[ch]
        ga.wait()
        gb.wait()
        lax.fori_loop(ch * nblk, (ch + 1) * nblk, body, 0)
        hsl = pl.ds(ch * half, half)
        writes.append(pltpu.async_copy(
            r0.at[hsl], out_hbm.at[pl.ds(base + ch * half, half)], semw))
    for wcp in writes:
        wcp.wait()


def _run_combine(ys, pos, gates):
    mesh = plsc.VectorSubcoreMesh(core_axis_name="c", subcore_axis_name="s")
    f = pl.kernel(
        _combine_body,
        mesh=mesh,
        out_type=jax.ShapeDtypeStruct((_T, _H), jnp.float32),
        scratch_types=[
            pltpu.VMEM((_TPW,), jnp.int32),
            pltpu.VMEM((_TPW,), jnp.int32),
            pltpu.VMEM((2, _TPW), jnp.float32),
            pltpu.VMEM((_TPW, _H), jnp.float32),
            pltpu.VMEM((_TPW, _H), jnp.float32),
            pltpu.SemaphoreType.DMA,
            pltpu.SemaphoreType.DMA,
            pltpu.SemaphoreType.DMA,
            pltpu.SemaphoreType.DMA,
        ],
    )
    return f(ys, pos, gates)


def kernel(hidden_states, Wr, br, W1, b1, W2, b2):
    B, S, Hd = hidden_states.shape
    x = hidden_states.reshape(S, Hd)
    w1b = W1
    w2b = W2
    b1r = b1.reshape(_E, 1, _F)
    b2r = b2.reshape(_E, 1, _H)

    pos, gates, bmeta = _run_router(x, Wr, br)
    xs = _run_dispatch(x, pos)
    ys = _run_ffn(bmeta, xs, w1b, b1r, w2b, b2r)
    out = _run_combine(ys, pos, gates)
    return out.reshape(B, S, Hd)


# speedup vs baseline: 1.0483x; 1.0134x over previous
import functools

import jax
import jax.numpy as jnp
from jax import lax
from jax.experimental import pallas as pl
from jax.experimental.pallas import tpu as pltpu
from jax.experimental.pallas import tpu_sc as plsc

_H = 768
_F = 2048
_E = 8
_T = 2048
_A = 2 * _T
_BT = 256
_NBMAX = _A // _BT + _E
_NP = _NBMAX * _BT
_NMETA = 32

_NW = 32
_APW = _A // _NW
_TPW = _T // _NW

_INV_SQRT2 = 0.7071067811865476


def _router_body(x_ref, wr_ref, br_ref, pos_ref, gates_ref, bmeta_ref,
                 csum_ref):
    x = x_ref[...]
    logits = jnp.dot(x, wr_ref[...], preferred_element_type=jnp.float32)
    logits = logits + br_ref[0, :]
    logits = logits - jnp.max(logits, axis=-1, keepdims=True)
    p = jnp.exp(logits)
    p = p / jnp.sum(p, axis=-1, keepdims=True)

    ids = lax.broadcasted_iota(jnp.int32, (_T, _E), 1)
    i1 = jnp.argmax(p, axis=-1)
    m1 = jnp.max(p, axis=-1, keepdims=True)
    oh1 = ids == i1[:, None]
    p2 = jnp.where(oh1, -jnp.inf, p)
    i2 = jnp.argmax(p2, axis=-1)
    m2 = jnp.max(p2, axis=-1, keepdims=True)
    oh2 = ids == i2[:, None]
    denom = m1 + m2
    gates_ref[0, :] = (m1 / denom)[:, 0]
    gates_ref[1, :] = (m2 / denom)[:, 0]

    maskcat = jnp.concatenate(
        [oh1.astype(jnp.bfloat16), oh2.astype(jnp.bfloat16)], axis=1)
    ti = lax.broadcasted_iota(jnp.int32, (128, 128), 0)
    tj = lax.broadcasted_iota(jnp.int32, (128, 128), 1)
    ltri = (ti > tj).astype(jnp.bfloat16)
    run8 = jnp.zeros((1, 2 * _E), jnp.float32)
    for i in range(_T // 128):
        chunk = maskcat[i * 128:(i + 1) * 128, :]
        c = jnp.dot(ltri, chunk, preferred_element_type=jnp.float32)
        csum_ref[pl.ds(i * 128, 128), :] = c + run8
        run8 = run8 + jnp.sum(chunk.astype(jnp.float32), axis=0,
                              keepdims=True)
    csum = csum_ref[...]
    total0 = run8[:, :_E]
    total1 = run8[:, _E:]
    rank0 = csum[:, :_E]
    rank1 = csum[:, _E:] + total0

    count = total0 + total1
    padded = jnp.ceil(count / _BT) * _BT
    ei = lax.broadcasted_iota(jnp.int32, (_E, _E), 0)
    ej = lax.broadcasted_iota(jnp.int32, (_E, _E), 1)
    utri = (ei < ej).astype(jnp.float32)
    pstart = jnp.dot(padded, utri,
                     preferred_element_type=jnp.float32)

    dest0 = jnp.sum(jnp.where(oh1, pstart + rank0, 0.0), axis=1)
    dest1 = jnp.sum(jnp.where(oh2, pstart + rank1, 0.0), axis=1)
    pos_ref[0, :] = dest0.astype(jnp.int32)
    pos_ref[1, :] = dest1.astype(jnp.int32)

    pend = (pstart + padded)[0, :]
    bidx = lax.broadcasted_iota(jnp.int32, (1, _NMETA), 1).astype(jnp.float32)
    cmp = (pend[:, None] <= bidx * _BT).astype(jnp.float32)
    be_raw = jnp.sum(cmp, axis=0, keepdims=True)
    nb = jnp.sum(padded) * (1.0 / _BT)
    lastexp = jnp.sum(jnp.where(bidx == nb - 1.0, be_raw, 0.0))
    bexp = jnp.minimum(be_raw, lastexp)
    meta = jnp.where(bidx < _NBMAX, bexp, nb)
    bmeta_ref[...] = meta.astype(jnp.int32)


def _run_router(x, Wr, br):
    return pl.pallas_call(
        _router_body,
        in_specs=[
            pl.BlockSpec((_T, _H), lambda: (0, 0)),
            pl.BlockSpec((_H, _E), lambda: (0, 0)),
            pl.BlockSpec((1, _E), lambda: (0, 0)),
        ],
        out_specs=[
            pl.BlockSpec((2, _T), lambda: (0, 0)),
            pl.BlockSpec((2, _T), lambda: (0, 0)),
            pl.BlockSpec((1, _NMETA), lambda: (0, 0)),
        ],
        out_shape=[
            jax.ShapeDtypeStruct((2, _T), jnp.int32),
            jax.ShapeDtypeStruct((2, _T), jnp.float32),
            jax.ShapeDtypeStruct((1, _NMETA), jnp.int32),
        ],
        scratch_shapes=[pltpu.VMEM((_T, 2 * _E), jnp.float32)],
    )(x, Wr, br.reshape(1, _E))


_BCH = 4
_BCR = _APW // _BCH


def _dispatch_body(x_hbm, pos_hbm, xs_hbm, idx_v, rows_v,
                   sem_i, sem_in, sem_out):
    c = lax.axis_index("c")
    s = lax.axis_index("s")
    wid = s * 2 + c
    k = wid // (_NW // 2)
    tokbase = (wid % (_NW // 2)) * _APW
    ics = []
    for ch in range(_BCH):
        ics.append(pltpu.async_copy(
            pos_hbm.at[k, pl.ds(tokbase + ch * _BCR, _BCR)],
            idx_v.at[ch], sem_i))
    loads = []
    for ch in range(_BCH):
        loads.append(pltpu.async_copy(
            x_hbm.at[pl.ds(tokbase + ch * _BCR, _BCR)],
            rows_v.at[pl.ds(ch * _BCR, _BCR)], sem_in[ch]))
    for ic in ics:
        ic.wait()
    scatters = []
    for ch in range(_BCH):
        loads[ch].wait()
        scatters.append(pltpu.async_copy(
            rows_v.at[pl.ds(ch * _BCR, _BCR)],
            xs_hbm.at[idx_v.at[ch]], sem_out))
    for sc in scatters:
        sc.wait()


def _run_dispatch(x, pos2d):
    mesh = plsc.VectorSubcoreMesh(core_axis_name="c", subcore_axis_name="s")
    f = pl.kernel(
        _dispatch_body,
        mesh=mesh,
        out_type=jax.ShapeDtypeStruct((_NP, _H), jnp.float32),
        scratch_types=[
            pltpu.VMEM((_BCH, _BCR), jnp.int32),
            pltpu.VMEM((_APW, _H), jnp.float32),
            pltpu.SemaphoreType.DMA,
            [pltpu.SemaphoreType.DMA] * _BCH,
            pltpu.SemaphoreType.DMA,
        ],
    )
    return f(x, pos2d)


def _ffn_body(bmeta_ref, x_ref, w1_ref, b1_ref, w2_ref, b2_ref, y_ref):
    b = pl.program_id(0)
    nb = bmeta_ref[0, _NBMAX]

    @pl.when(b < nb)
    def _compute():
        xb = x_ref[...].astype(jnp.bfloat16)
        h = jnp.dot(xb, w1_ref[0].astype(jnp.bfloat16),
                    preferred_element_type=jnp.float32)
        h = h + b1_ref[0]
        h = 0.5 * h * (1.0 + lax.erf(h * _INV_SQRT2))
        y = jnp.dot(h.astype(jnp.bfloat16), w2_ref[0].astype(jnp.bfloat16),
                    preferred_element_type=jnp.float32)
        y_ref[...] = y + b2_ref[0]


def _run_ffn(bmeta, xs, w1b, b1r, w2b, b2r):
    grid_spec = pltpu.PrefetchScalarGridSpec(
        num_scalar_prefetch=1,
        grid=(_NBMAX,),
        in_specs=[
            pl.BlockSpec((_BT, _H),
                         lambda b, m: (jnp.where(b < m[0, _NBMAX], b, 0), 0)),
            pl.BlockSpec((1, _H, _F), lambda b, m: (m[0, b], 0, 0)),
            pl.BlockSpec((1, 1, _F), lambda b, m: (m[0, b], 0, 0)),
            pl.BlockSpec((1, _F, _H), lambda b, m: (m[0, b], 0, 0)),
            pl.BlockSpec((1, 1, _H), lambda b, m: (m[0, b], 0, 0)),
        ],
        out_specs=pl.BlockSpec((_BT, _H), lambda b, m: (b, 0)),
    )
    return pl.pallas_call(
        _ffn_body,
        grid_spec=grid_spec,
        out_shape=jax.ShapeDtypeStruct((_NP, _H), jnp.float32),
        compiler_params=pltpu.CompilerParams(
            dimension_semantics=("arbitrary",),
        ),
    )(bmeta, xs, w1b, b1r, w2b, b2r)


def _combine_body(y_hbm, pos_hbm, gates_hbm, out_hbm,
                  idx0, idx1, g_vm, r0, r1, sem, sem0, sem1, semw):
    c = lax.axis_index("c")
    s = lax.axis_index("s")
    wid = s * 2 + c
    base = wid * _TPW
    c0 = pltpu.async_copy(pos_hbm.at[0, pl.ds(base, _TPW)], idx0, sem)
    c1 = pltpu.async_copy(pos_hbm.at[1, pl.ds(base, _TPW)], idx1, sem)
    c2 = pltpu.async_copy(gates_hbm.at[0, pl.ds(base, _TPW)], g_vm.at[0],
                          sem)
    c3 = pltpu.async_copy(gates_hbm.at[1, pl.ds(base, _TPW)], g_vm.at[1],
                          sem)
    c0.wait(); c1.wait(); c2.wait(); c3.wait()

    half = _TPW // 2
    gsems = (sem0, sem1)
    gathers = []
    for ch in range(2):
        hsl = pl.ds(ch * half, half)
        gathers.append((
            pltpu.async_copy(y_hbm.at[idx0.at[hsl]], r0.at[hsl], gsems[ch]),
            pltpu.async_copy(y_hbm.at[idx1.at[hsl]], r1.at[hsl], gsems[ch]),
        ))

    def body(blk, _):
        gv0 = g_vm[0, pl.ds(blk * 16, 16)]
        gv1 = g_vm[1, pl.ds(blk * 16, 16)]
        for ii in range(16):
            g0 = gv0[ii]
            g1 = gv1[ii]
            row = blk * 16 + ii
            for j in range(_H // 16):
                sl = pl.ds(j * 16, 16)
                r0[row, sl] = r0[row, sl] * g0 + r1[row, sl] * g1
        return 0

    writes = []
    nblk = half // 16
    for ch in range(2):
        ga, gb = gathers[ch]
        ga.wait()
        gb.wait()
        lax.fori_loop(ch * nblk, (ch + 1) * nblk, body, 0)
        hsl = pl.ds(ch * half, half)
        writes.append(pltpu.async_copy(
            r0.at[hsl], out_hbm.at[pl.ds(base + ch * half, half)], semw))
    for wcp in writes:
        wcp.wait()


def _run_combine(ys, pos, gates):
    mesh = plsc.VectorSubcoreMesh(core_axis_name="c", subcore_axis_name="s")
    f = pl.kernel(
        _combine_body,
        mesh=mesh,
        out_type=jax.ShapeDtypeStruct((_T, _H), jnp.float32),
        scratch_types=[
            pltpu.VMEM((_TPW,), jnp.int32),
            pltpu.VMEM((_TPW,), jnp.int32),
            pltpu.VMEM((2, _TPW), jnp.float32),
            pltpu.VMEM((_TPW, _H), jnp.float32),
            pltpu.VMEM((_TPW, _H), jnp.float32),
            pltpu.SemaphoreType.DMA,
            pltpu.SemaphoreType.DMA,
            pltpu.SemaphoreType.DMA,
            pltpu.SemaphoreType.DMA,
        ],
    )
    return f(ys, pos, gates)


def kernel(hidden_states, Wr, br, W1, b1, W2, b2):
    B, S, Hd = hidden_states.shape
    x = hidden_states.reshape(S, Hd)
    w1b = W1
    w2b = W2
    b1r = b1.reshape(_E, 1, _F)
    b2r = b2.reshape(_E, 1, _H)

    pos, gates, bmeta = _run_router(x, Wr, br)
    xs = _run_dispatch(x, pos)
    ys = _run_ffn(bmeta, xs, w1b, b1r, w2b, b2r)
    out = _run_combine(ys, pos, gates)
    return out.reshape(B, S, Hd)


# speedup vs baseline: 1.0530x; 1.0046x over previous
import jax
import jax.numpy as jnp
from jax import lax
from jax.experimental import pallas as pl
from jax.experimental.pallas import tpu as pltpu
from jax.experimental.pallas import tpu_sc as plsc

_H = 768
_F = 2048
_E = 8
_T = 2048
_A = 2 * _T
_BT = 256
_NBMAX = _A // _BT + _E
_NP = _NBMAX * _BT
_NMETA = 32

_NW = 32
_APW = _A // _NW
_TPW = _T // _NW

_INV_SQRT2 = 0.7071067811865476


def _router_body(x_ref, wr_ref, br_ref, pos_ref, gates_ref, bmeta_ref,
                 csum_ref):
    x = x_ref[...]
    logits = jnp.dot(x, wr_ref[...], preferred_element_type=jnp.float32)
    logits = logits + br_ref[0, :]
    logits = logits - jnp.max(logits, axis=-1, keepdims=True)
    p = jnp.exp(logits)
    p = p / jnp.sum(p, axis=-1, keepdims=True)

    ids = lax.broadcasted_iota(jnp.int32, (_T, _E), 1)
    i1 = jnp.argmax(p, axis=-1)
    m1 = jnp.max(p, axis=-1, keepdims=True)
    oh1 = ids == i1[:, None]
    p2 = jnp.where(oh1, -jnp.inf, p)
    i2 = jnp.argmax(p2, axis=-1)
    m2 = jnp.max(p2, axis=-1, keepdims=True)
    oh2 = ids == i2[:, None]
    denom = m1 + m2
    gates_ref[0, :] = (m1 / denom)[:, 0]
    gates_ref[1, :] = (m2 / denom)[:, 0]

    maskcat = jnp.concatenate(
        [oh1.astype(jnp.bfloat16), oh2.astype(jnp.bfloat16)], axis=1)
    ti = lax.broadcasted_iota(jnp.int32, (128, 128), 0)
    tj = lax.broadcasted_iota(jnp.int32, (128, 128), 1)
    ltri = (ti > tj).astype(jnp.bfloat16)
    run8 = jnp.zeros((1, 2 * _E), jnp.float32)
    for i in range(_T // 128):
        chunk = maskcat[i * 128:(i + 1) * 128, :]
        c = jnp.dot(ltri, chunk, preferred_element_type=jnp.float32)
        csum_ref[pl.ds(i * 128, 128), :] = c + run8
        run8 = run8 + jnp.sum(chunk.astype(jnp.float32), axis=0,
                              keepdims=True)
    csum = csum_ref[...]
    total0 = run8[:, :_E]
    total1 = run8[:, _E:]
    rank0 = csum[:, :_E]
    rank1 = csum[:, _E:] + total0

    count = total0 + total1
    padded = jnp.ceil(count / _BT) * _BT
    ei = lax.broadcasted_iota(jnp.int32, (_E, _E), 0)
    ej = lax.broadcasted_iota(jnp.int32, (_E, _E), 1)
    utri = (ei < ej).astype(jnp.float32)
    pstart = jnp.dot(padded, utri,
                     preferred_element_type=jnp.float32)

    dest0 = jnp.sum(jnp.where(oh1, pstart + rank0, 0.0), axis=1)
    dest1 = jnp.sum(jnp.where(oh2, pstart + rank1, 0.0), axis=1)
    pos_ref[0, :] = dest0.astype(jnp.int32)
    pos_ref[1, :] = dest1.astype(jnp.int32)

    pend = (pstart + padded)[0, :]
    bidx = lax.broadcasted_iota(jnp.int32, (1, _NMETA), 1).astype(jnp.float32)
    cmp = (pend[:, None] <= bidx * _BT).astype(jnp.float32)
    be_raw = jnp.sum(cmp, axis=0, keepdims=True)
    nb = jnp.sum(padded) * (1.0 / _BT)
    lastexp = jnp.sum(jnp.where(bidx == nb - 1.0, be_raw, 0.0))
    bexp = jnp.minimum(be_raw, lastexp)
    meta = jnp.where(bidx < _NBMAX, bexp, nb)
    bmeta_ref[...] = meta.astype(jnp.int32)


def _run_router(x, Wr, br):
    return pl.pallas_call(
        _router_body,
        in_specs=[
            pl.BlockSpec((_T, _H), lambda: (0, 0)),
            pl.BlockSpec((_H, _E), lambda: (0, 0)),
            pl.BlockSpec((1, _E), lambda: (0, 0)),
        ],
        out_specs=[
            pl.BlockSpec((2, _T), lambda: (0, 0)),
            pl.BlockSpec((2, _T), lambda: (0, 0)),
            pl.BlockSpec((1, _NMETA), lambda: (0, 0)),
        ],
        out_shape=[
            jax.ShapeDtypeStruct((2, _T), jnp.int32),
            jax.ShapeDtypeStruct((2, _T), jnp.float32),
            jax.ShapeDtypeStruct((1, _NMETA), jnp.int32),
        ],
        scratch_shapes=[pltpu.VMEM((_T, 2 * _E), jnp.float32)],
    )(x, Wr, br.reshape(1, _E))


_BCH = 4
_BCR = _APW // _BCH


def _dispatch_body(x_hbm, pos_hbm, xs_hbm, idx_v, rows_v,
                   sem_i, sem_in, sem_out):
    c = lax.axis_index("c")
    s = lax.axis_index("s")
    wid = s * 2 + c
    k = wid // (_NW // 2)
    tokbase = (wid % (_NW // 2)) * _APW
    ics = []
    for ch in range(_BCH):
        ics.append(pltpu.async_copy(
            pos_hbm.at[k, pl.ds(tokbase + ch * _BCR, _BCR)],
            idx_v.at[ch], sem_i))
    loads = []
    for ch in range(_BCH):
        loads.append(pltpu.async_copy(
            x_hbm.at[pl.ds(tokbase + ch * _BCR, _BCR)],
            rows_v.at[pl.ds(ch * _BCR, _BCR)], sem_in[ch]))
    for ic in ics:
        ic.wait()
    scatters = []
    for ch in range(_BCH):
        loads[ch].wait()
        scatters.append(pltpu.async_copy(
            rows_v.at[pl.ds(ch * _BCR, _BCR)],
            xs_hbm.at[idx_v.at[ch]], sem_out))
    for sc in scatters:
        sc.wait()


def _run_dispatch(x, pos2d):
    mesh = plsc.VectorSubcoreMesh(core_axis_name="c", subcore_axis_name="s")
    f = pl.kernel(
        _dispatch_body,
        mesh=mesh,
        out_type=jax.ShapeDtypeStruct((_NP, _H), jnp.float32),
        scratch_types=[
            pltpu.VMEM((_BCH, _BCR), jnp.int32),
            pltpu.VMEM((_APW, _H), jnp.float32),
            pltpu.SemaphoreType.DMA,
            [pltpu.SemaphoreType.DMA] * _BCH,
            pltpu.SemaphoreType.DMA,
        ],
    )
    return f(x, pos2d)


def _ffn_body(bmeta_ref, x_ref, w1_ref, b1_ref, w2_ref, b2_ref, y_ref):
    b = pl.program_id(0)
    nb = bmeta_ref[0, _NBMAX]

    @pl.when(b < nb)
    def _compute():
        xb = x_ref[...].astype(jnp.bfloat16)
        h = jnp.dot(xb, w1_ref[0].astype(jnp.bfloat16),
                    preferred_element_type=jnp.float32)
        h = h + b1_ref[0]
        h = 0.5 * h * (1.0 + lax.erf(h * _INV_SQRT2))
        y = jnp.dot(h.astype(jnp.bfloat16), w2_ref[0].astype(jnp.bfloat16),
                    preferred_element_type=jnp.float32)
        y_ref[...] = y + b2_ref[0]


def _run_ffn(bmeta, xs, w1b, b1r, w2b, b2r):
    grid_spec = pltpu.PrefetchScalarGridSpec(
        num_scalar_prefetch=1,
        grid=(_NBMAX,),
        in_specs=[
            pl.BlockSpec((_BT, _H),
                         lambda b, m: (jnp.where(b < m[0, _NBMAX], b, 0), 0)),
            pl.BlockSpec((1, _H, _F), lambda b, m: (m[0, b], 0, 0)),
            pl.BlockSpec((1, 1, _F), lambda b, m: (m[0, b], 0, 0)),
            pl.BlockSpec((1, _F, _H), lambda b, m: (m[0, b], 0, 0)),
            pl.BlockSpec((1, 1, _H), lambda b, m: (m[0, b], 0, 0)),
        ],
        out_specs=pl.BlockSpec((_BT, _H), lambda b, m: (b, 0)),
    )
    return pl.pallas_call(
        _ffn_body,
        grid_spec=grid_spec,
        out_shape=jax.ShapeDtypeStruct((_NP, _H), jnp.float32),
        compiler_params=pltpu.CompilerParams(
            dimension_semantics=("arbitrary",),
        ),
    )(bmeta, xs, w1b, b1r, w2b, b2r)


def _combine_body(y_hbm, pos_hbm, gates_hbm, out_hbm,
                  idx0, idx1, g_vm, r0, r1, sem, sem0, sem1, semw):
    c = lax.axis_index("c")
    s = lax.axis_index("s")
    wid = s * 2 + c
    base = wid * _TPW
    c0 = pltpu.async_copy(pos_hbm.at[0, pl.ds(base, _TPW)], idx0, sem)
    c1 = pltpu.async_copy(pos_hbm.at[1, pl.ds(base, _TPW)], idx1, sem)
    c2 = pltpu.async_copy(gates_hbm.at[0, pl.ds(base, _TPW)], g_vm.at[0],
                          sem)
    c3 = pltpu.async_copy(gates_hbm.at[1, pl.ds(base, _TPW)], g_vm.at[1],
                          sem)
    c0.wait(); c1.wait(); c2.wait(); c3.wait()

    half = _TPW // 2
    gsems = (sem0, sem1)
    gathers = []
    for ch in range(2):
        hsl = pl.ds(ch * half, half)
        gathers.append((
            pltpu.async_copy(y_hbm.at[idx0.at[hsl]], r0.at[hsl], gsems[ch]),
            pltpu.async_copy(y_hbm.at[idx1.at[hsl]], r1.at[hsl], gsems[ch]),
        ))

    def body(blk, _):
        gv0 = g_vm[0, pl.ds(blk * 16, 16)]
        gv1 = g_vm[1, pl.ds(blk * 16, 16)]
        for ii in range(16):
            g0 = gv0[ii]
            g1 = gv1[ii]
            row = blk * 16 + ii
            for j in range(_H // 16):
                sl = pl.ds(j * 16, 16)
                r0[row, sl] = r0[row, sl] * g0 + r1[row, sl] * g1
        return 0

    writes = []
    nblk = half // 16
    for ch in range(2):
        ga, gb = gathers[ch]
        ga.wait()
        gb.wait()
        lax.fori_loop(ch * nblk, (ch + 1) * nblk, body, 0)
        hsl = pl.ds(ch * half, half)
        writes.append(pltpu.async_copy(
            r0.at[hsl], out_hbm.at[pl.ds(base + ch * half, half)], semw))
    for wcp in writes:
        wcp.wait()


def _run_combine(ys, pos, gates):
    mesh = plsc.VectorSubcoreMesh(core_axis_name="c", subcore_axis_name="s")
    f = pl.kernel(
        _combine_body,
        mesh=mesh,
        out_type=jax.ShapeDtypeStruct((_T, _H), jnp.float32),
        scratch_types=[
            pltpu.VMEM((_TPW,), jnp.int32),
            pltpu.VMEM((_TPW,), jnp.int32),
            pltpu.VMEM((2, _TPW), jnp.float32),
            pltpu.VMEM((_TPW, _H), jnp.float32),
            pltpu.VMEM((_TPW, _H), jnp.float32),
            pltpu.SemaphoreType.DMA,
            pltpu.SemaphoreType.DMA,
            pltpu.SemaphoreType.DMA,
            pltpu.SemaphoreType.DMA,
        ],
    )
    return f(ys, pos, gates)


def kernel(hidden_states, Wr, br, W1, b1, W2, b2):
    B, S, Hd = hidden_states.shape
    x = hidden_states.reshape(S, Hd)
    b1r = b1.reshape(_E, 1, _F)
    b2r = b2.reshape(_E, 1, _H)

    pos, gates, bmeta = _run_router(x, Wr, br)
    xs = _run_dispatch(x, pos)
    ys = _run_ffn(bmeta, xs, W1, b1r, W2, b2r)
    out = _run_combine(ys, pos, gates)
    return out.reshape(B, S, Hd)
